# baseline (speedup 1.0000x reference)
import jax
import jax.numpy as jnp
from jax import lax
from jax.experimental import pallas as pl
from jax.experimental.pallas import tpu as pltpu
from jax.experimental.pallas import tpu_sc as plsc

NC, NS = 2, 16
NW = NC * NS
C = 1600
G = 80
NG = C // G
LANES = 16
STEPS = C // LANES


def _body(xs_hbm, ys_hbm, zs_hbm, ii_hbm, jj_hbm, ox_hbm, oy_hbm, oz_hbm,
          out_hbm, xs_sh, ys_sh, zs_sh, *bufs):
  per_w = ii_hbm.shape[0] // NW
  n_chunks = per_w // C

  cid = lax.axis_index("c")
  sid = lax.axis_index("s")
  wid = sid * NC + cid

  (ii_a, jj_a, ox_a, oy_a, oz_a, xi_a, yi_a, zi_a, xj_a, yj_a, zj_a, out_a,
   lsem_a, gsem_a, osem_a,
   ii_b, jj_b, ox_b, oy_b, oz_b, xi_b, yi_b, zi_b, xj_b, yj_b, zj_b, out_b,
   lsem_b, gsem_b, osem_b) = bufs
  A = (ii_a, jj_a, (ox_a, oy_a, oz_a), (xi_a, yi_a, zi_a),
       (xj_a, yj_a, zj_a), out_a, lsem_a, gsem_a, osem_a)
  B = (ii_b, jj_b, (ox_b, oy_b, oz_b), (xi_b, yi_b, zi_b),
       (xj_b, yj_b, zj_b), out_b, lsem_b, gsem_b, osem_b)

  @pl.when(sid == 0)
  def _():
    pltpu.sync_copy(xs_hbm, xs_sh)
    pltpu.sync_copy(ys_hbm, ys_sh)
    pltpu.sync_copy(zs_hbm, zs_sh)
  plsc.subcore_barrier()

  tabs = (xs_sh, ys_sh, zs_sh)
  offs_hbm = (ox_hbm, oy_hbm, oz_hbm)

  def chunk_base(g):
    return wid * per_w + g * C

  def lin_start(g, S):
    ii_v, jj_v, ov, _, _, _, lsem, _, _ = S
    @pl.when(g < n_chunks)
    def _():
      base = chunk_base(g)
      sl = pl.ds(base, C)
      pltpu.async_copy(ii_hbm.at[sl], ii_v, lsem)
      pltpu.async_copy(jj_hbm.at[sl], jj_v, lsem)
      for t in range(3):
        pltpu.async_copy(offs_hbm[t].at[sl], ov[t], lsem)

  def lin_wait(S):
    ii_v, jj_v, ov, _, _, _, lsem, _, _ = S
    pltpu.make_async_copy(ii_hbm.at[pl.ds(0, C)], ii_v, lsem).wait()
    pltpu.make_async_copy(jj_hbm.at[pl.ds(0, C)], jj_v, lsem).wait()
    for t in range(3):
      pltpu.make_async_copy(ox_hbm.at[pl.ds(0, C)], ov[t], lsem).wait()

  def gather_fire(S):
    ii_v, jj_v, _, ri, rj, _, _, gsem, _ = S
    def fire(k, c2):
      sl = pl.ds(k * G, G)
      for t in range(3):
        pltpu.async_copy(tabs[t].at[ii_v.at[sl]], ri[t].at[sl], gsem)
        pltpu.async_copy(tabs[t].at[jj_v.at[sl]], rj[t].at[sl], gsem)
      return c2
    lax.fori_loop(0, NG, fire, 0)

  def gather_drain(S):
    _, _, _, ri, rj, _, _, gsem, _ = S
    for buf in (*ri, *rj):
      pltpu.make_async_copy(xs_hbm.at[pl.ds(0, C)], buf, gsem).wait()

  def out_wait(S):
    out_v, osem = S[5], S[8]
    pltpu.make_async_copy(out_hbm.at[pl.ds(0, C)], out_v, osem).wait()

  def compute_store(g, S):
    _, _, (ox_v, oy_v, oz_v), (xi_v, yi_v, zi_v), (xj_v, yj_v, zj_v), \
        out_v, _, _, osem = S
    def step(s, c2):
      sl = pl.ds(s * LANES, LANES)
      dx = xi_v[sl] - xj_v[sl] - ox_v[sl]
      dy = yi_v[sl] - yj_v[sl] - oy_v[sl]
      dz = zi_v[sl] - zj_v[sl] - oz_v[sl]
      d2 = jnp.maximum(dx * dx + dy * dy + dz * dz, 0.0)
      seed = plsc.bitcast(0x5F3759DF - (plsc.bitcast(d2, jnp.int32) >> 1),
                          jnp.float32)
      r = seed * (1.5 - 0.5 * d2 * seed * seed)
      out_v[sl] = d2 * r
      return c2
    lax.fori_loop(0, STEPS, step, 0)
    pltpu.async_copy(out_v, out_hbm.at[pl.ds(chunk_base(g), C)], osem)

  pltpu.async_copy(out_hbm.at[pl.ds(0, C)], A[5], A[8])
  pltpu.async_copy(out_hbm.at[pl.ds(0, C)], B[5], B[8])

  lin_start(0, A)
  lin_wait(A)
  gather_fire(A)
  lin_start(1, B)

  def pair(t, carry):
    g = 2 * t
    gather_drain(A)
    lin_wait(B)
    gather_fire(B)
    out_wait(A)
    compute_store(g, A)
    lin_start(g + 2, A)
    gather_drain(B)
    @pl.when(g + 2 < n_chunks)
    def _():
      lin_wait(A)
      gather_fire(A)
    out_wait(B)
    compute_store(g + 1, B)
    lin_start(g + 3, B)
    return carry

  lax.fori_loop(0, (n_chunks - 1) // 2, pair, 0)

  gather_drain(A)
  out_wait(A)
  compute_store(n_chunks - 1, A)
  out_wait(A)
  out_wait(B)


def kernel(Ra, idx_i, idx_j, offsets):
  n = Ra.shape[0]
  e = idx_i.shape[0]
  xs = Ra[:, 0]
  ys = Ra[:, 1]
  zs = Ra[:, 2]

  ox = offsets[:, 0]
  oy = offsets[:, 1]
  oz = offsets[:, 2]

  mesh = plsc.VectorSubcoreMesh(core_axis_name="c", subcore_axis_name="s")
  buf_set = [
      pltpu.VMEM((C,), jnp.int32),
      pltpu.VMEM((C,), jnp.int32),
      pltpu.VMEM((C,), jnp.float32),
      pltpu.VMEM((C,), jnp.float32),
      pltpu.VMEM((C,), jnp.float32),
      pltpu.VMEM((C,), jnp.float32),
      pltpu.VMEM((C,), jnp.float32),
      pltpu.VMEM((C,), jnp.float32),
      pltpu.VMEM((C,), jnp.float32),
      pltpu.VMEM((C,), jnp.float32),
      pltpu.VMEM((C,), jnp.float32),
      pltpu.VMEM((C,), jnp.float32),
      pltpu.SemaphoreType.DMA,
      pltpu.SemaphoreType.DMA,
      pltpu.SemaphoreType.DMA,
  ]
  run = pl.kernel(
      _body,
      out_type=jax.ShapeDtypeStruct((e,), jnp.float32),
      mesh=mesh,
      compiler_params=pltpu.CompilerParams(needs_layout_passes=False),
      scratch_types=[
          pltpu.VMEM_SHARED((n,), jnp.float32),
          pltpu.VMEM_SHARED((n,), jnp.float32),
          pltpu.VMEM_SHARED((n,), jnp.float32),
          *buf_set,
          *buf_set,
      ],
  )
  return run(xs, ys, zs, idx_i.astype(jnp.int32), idx_j.astype(jnp.int32),
             ox, oy, oz)


# speedup vs baseline: 1.0226x; 1.0226x over previous
import jax
import jax.numpy as jnp
from jax import lax
from jax.experimental import pallas as pl
from jax.experimental.pallas import tpu as pltpu
from jax.experimental.pallas import tpu_sc as plsc

NC, NS = 2, 16
NW = NC * NS
C = 1600
G = 80
NG = C // G
LANES = 16
STEPS = C // LANES


def _body(xs_hbm, ys_hbm, zs_hbm, ii_hbm, jj_hbm, ox_hbm, oy_hbm, oz_hbm,
          out_hbm, xs_sh, ys_sh, zs_sh, *bufs):
  per_w = ii_hbm.shape[0] // NW
  n_chunks = per_w // C

  cid = lax.axis_index("c")
  sid = lax.axis_index("s")
  wid = sid * NC + cid

  (ii_a, jj_a, ox_a, oy_a, oz_a, xi_a, yi_a, zi_a, xj_a, yj_a, zj_a, out_a,
   lsem_a, gsem_a, osem_a,
   ii_b, jj_b, ox_b, oy_b, oz_b, xi_b, yi_b, zi_b, xj_b, yj_b, zj_b, out_b,
   lsem_b, gsem_b, osem_b) = bufs
  A = (ii_a, jj_a, (ox_a, oy_a, oz_a), (xi_a, yi_a, zi_a),
       (xj_a, yj_a, zj_a), out_a, lsem_a, gsem_a, osem_a)
  B = (ii_b, jj_b, (ox_b, oy_b, oz_b), (xi_b, yi_b, zi_b),
       (xj_b, yj_b, zj_b), out_b, lsem_b, gsem_b, osem_b)

  @pl.when(sid == 0)
  def _():
    pltpu.sync_copy(xs_hbm, xs_sh)
    pltpu.sync_copy(ys_hbm, ys_sh)
    pltpu.sync_copy(zs_hbm, zs_sh)
  plsc.subcore_barrier()

  tabs = (xs_sh, ys_sh, zs_sh)
  offs_hbm = (ox_hbm, oy_hbm, oz_hbm)

  def chunk_base(g):
    return wid * per_w + g * C

  def lin_start(g, S):
    ii_v, jj_v, ov, _, _, _, lsem, _, _ = S
    @pl.when(g < n_chunks)
    def _():
      base = chunk_base(g)
      sl = pl.ds(base, C)
      pltpu.async_copy(ii_hbm.at[sl], ii_v, lsem)
      pltpu.async_copy(jj_hbm.at[sl], jj_v, lsem)
      for t in range(3):
        pltpu.async_copy(offs_hbm[t].at[sl], ov[t], lsem)

  def lin_wait(S):
    ii_v, jj_v, ov, _, _, _, lsem, _, _ = S
    pltpu.make_async_copy(ii_hbm.at[pl.ds(0, C)], ii_v, lsem).wait()
    pltpu.make_async_copy(jj_hbm.at[pl.ds(0, C)], jj_v, lsem).wait()
    for t in range(3):
      pltpu.make_async_copy(ox_hbm.at[pl.ds(0, C)], ov[t], lsem).wait()

  def gather_fire(S):
    ii_v, jj_v, _, ri, rj, _, _, gsem, _ = S
    def fire(k, c2):
      sl = pl.ds(k * G, G)
      for t in range(3):
        pltpu.async_copy(tabs[t].at[ii_v.at[sl]], ri[t].at[sl], gsem)
        pltpu.async_copy(tabs[t].at[jj_v.at[sl]], rj[t].at[sl], gsem)
      return c2
    lax.fori_loop(0, NG, fire, 0)

  def gather_drain(S):
    _, _, _, ri, rj, _, _, gsem, _ = S
    for buf in (*ri, *rj):
      pltpu.make_async_copy(xs_hbm.at[pl.ds(0, C)], buf, gsem).wait()

  def out_wait(S):
    out_v, osem = S[5], S[8]
    pltpu.make_async_copy(out_hbm.at[pl.ds(0, C)], out_v, osem).wait()

  def compute_store(g, S):
    _, _, (ox_v, oy_v, oz_v), (xi_v, yi_v, zi_v), (xj_v, yj_v, zj_v), \
        out_v, _, _, osem = S
    def step(s, c2):
      sl = pl.ds(s * LANES, LANES)
      dx = xi_v[sl] - xj_v[sl] - ox_v[sl]
      dy = yi_v[sl] - yj_v[sl] - oy_v[sl]
      dz = zi_v[sl] - zj_v[sl] - oz_v[sl]
      d2 = jnp.maximum(dx * dx + dy * dy + dz * dz, 0.0)
      seed = plsc.bitcast(0x5F3759DF - (plsc.bitcast(d2, jnp.int32) >> 1),
                          jnp.float32)
      h = 0.5 * d2
      r = seed * (1.5 - h * seed * seed)
      r = r * (1.5 - h * r * r)
      out_v[sl] = d2 * r
      return c2
    lax.fori_loop(0, STEPS, step, 0)
    pltpu.async_copy(out_v, out_hbm.at[pl.ds(chunk_base(g), C)], osem)

  pltpu.async_copy(out_hbm.at[pl.ds(0, C)], A[5], A[8])
  pltpu.async_copy(out_hbm.at[pl.ds(0, C)], B[5], B[8])

  lin_start(0, A)
  lin_wait(A)
  gather_fire(A)
  lin_start(1, B)

  def pair(t, carry):
    g = 2 * t
    gather_drain(A)
    lin_wait(B)
    gather_fire(B)
    out_wait(A)
    compute_store(g, A)
    lin_start(g + 2, A)
    gather_drain(B)
    @pl.when(g + 2 < n_chunks)
    def _():
      lin_wait(A)
      gather_fire(A)
    out_wait(B)
    compute_store(g + 1, B)
    lin_start(g + 3, B)
    return carry

  lax.fori_loop(0, (n_chunks - 1) // 2, pair, 0)

  gather_drain(A)
  out_wait(A)
  compute_store(n_chunks - 1, A)
  out_wait(A)
  out_wait(B)


def kernel(Ra, idx_i, idx_j, offsets):
  n = Ra.shape[0]
  e = idx_i.shape[0]
  xs = Ra[:, 0]
  ys = Ra[:, 1]
  zs = Ra[:, 2]

  ox = offsets[:, 0]
  oy = offsets[:, 1]
  oz = offsets[:, 2]

  mesh = plsc.VectorSubcoreMesh(core_axis_name="c", subcore_axis_name="s")
  buf_set = [
      pltpu.VMEM((C,), jnp.int32),
      pltpu.VMEM((C,), jnp.int32),
      pltpu.VMEM((C,), jnp.float32),
      pltpu.VMEM((C,), jnp.float32),
      pltpu.VMEM((C,), jnp.float32),
      pltpu.VMEM((C,), jnp.float32),
      pltpu.VMEM((C,), jnp.float32),
      pltpu.VMEM((C,), jnp.float32),
      pltpu.VMEM((C,), jnp.float32),
      pltpu.VMEM((C,), jnp.float32),
      pltpu.VMEM((C,), jnp.float32),
      pltpu.VMEM((C,), jnp.float32),
      pltpu.SemaphoreType.DMA,
      pltpu.SemaphoreType.DMA,
      pltpu.SemaphoreType.DMA,
  ]
  run = pl.kernel(
      _body,
      out_type=jax.ShapeDtypeStruct((e,), jnp.float32),
      mesh=mesh,
      compiler_params=pltpu.CompilerParams(needs_layout_passes=False),
      scratch_types=[
          pltpu.VMEM_SHARED((n,), jnp.float32),
          pltpu.VMEM_SHARED((n,), jnp.float32),
          pltpu.VMEM_SHARED((n,), jnp.float32),
          *buf_set,
          *buf_set,
      ],
  )
  return run(xs, ys, zs, idx_i.astype(jnp.int32), idx_j.astype(jnp.int32),
             ox, oy, oz)


# speedup vs baseline: 1.0245x; 1.0018x over previous
import jax
import jax.numpy as jnp
from jax import lax
from jax.experimental import pallas as pl
from jax.experimental.pallas import tpu as pltpu
from jax.experimental.pallas import tpu_sc as plsc

NC, NS = 2, 16
NW = NC * NS
C = 1600
G = 80
NG = C // G
LANES = 16
STEPS = C // LANES


def _body(xs_hbm, ys_hbm, zs_hbm, ii_hbm, jj_hbm, ox_hbm, oy_hbm, oz_hbm,
          out_hbm, xs_sh, ys_sh, zs_sh, *bufs):
  per_w = ii_hbm.shape[0] // NW
  n_chunks = per_w // C

  cid = lax.axis_index("c")
  sid = lax.axis_index("s")
  wid = sid * NC + cid

  (ii_a, jj_a, ox_a, oy_a, oz_a, xi_a, yi_a, zi_a, xj_a, yj_a, zj_a, out_a,
   lsem_a, gsem_a, osem_a,
   ii_b, jj_b, ox_b, oy_b, oz_b, xi_b, yi_b, zi_b, xj_b, yj_b, zj_b, out_b,
   lsem_b, gsem_b, osem_b) = bufs
  A = (ii_a, jj_a, (ox_a, oy_a, oz_a), (xi_a, yi_a, zi_a),
       (xj_a, yj_a, zj_a), out_a, lsem_a, gsem_a, osem_a)
  B = (ii_b, jj_b, (ox_b, oy_b, oz_b), (xi_b, yi_b, zi_b),
       (xj_b, yj_b, zj_b), out_b, lsem_b, gsem_b, osem_b)

  @pl.when(sid == 0)
  def _():
    pltpu.sync_copy(xs_hbm, xs_sh)
    pltpu.sync_copy(ys_hbm, ys_sh)
    pltpu.sync_copy(zs_hbm, zs_sh)
  plsc.subcore_barrier()

  tabs = (xs_sh, ys_sh, zs_sh)
  offs_hbm = (ox_hbm, oy_hbm, oz_hbm)

  def chunk_base(g):
    return wid * per_w + g * C

  def lin_start(g, S):
    ii_v, jj_v, ov, _, _, _, lsem, _, _ = S
    @pl.when(g < n_chunks)
    def _():
      base = chunk_base(g)
      sl = pl.ds(base, C)
      pltpu.async_copy(ii_hbm.at[sl], ii_v, lsem)
      pltpu.async_copy(jj_hbm.at[sl], jj_v, lsem)
      for t in range(3):
        pltpu.async_copy(offs_hbm[t].at[sl], ov[t], lsem)

  def lin_wait(S):
    ii_v, jj_v, ov, _, _, _, lsem, _, _ = S
    pltpu.make_async_copy(ii_hbm.at[pl.ds(0, C)], ii_v, lsem).wait()
    pltpu.make_async_copy(jj_hbm.at[pl.ds(0, C)], jj_v, lsem).wait()
    for t in range(3):
      pltpu.make_async_copy(ox_hbm.at[pl.ds(0, C)], ov[t], lsem).wait()

  def gather_fire(S):
    ii_v, jj_v, _, ri, rj, _, _, gsem, _ = S
    def fire(k, c2):
      sl = pl.ds(k * G, G)
      for t in range(3):
        pltpu.async_copy(tabs[t].at[ii_v.at[sl]], ri[t].at[sl], gsem)
        pltpu.async_copy(tabs[t].at[jj_v.at[sl]], rj[t].at[sl], gsem)
      return c2
    lax.fori_loop(0, NG, fire, 0)

  def gather_drain(S):
    _, _, _, ri, rj, _, _, gsem, _ = S
    for buf in (*ri, *rj):
      pltpu.make_async_copy(xs_hbm.at[pl.ds(0, C)], buf, gsem).wait()

  def out_wait(S):
    out_v, osem = S[5], S[8]
    pltpu.make_async_copy(out_hbm.at[pl.ds(0, C)], out_v, osem).wait()

  def compute_store(g, S):
    _, _, (ox_v, oy_v, oz_v), (xi_v, yi_v, zi_v), (xj_v, yj_v, zj_v), \
        out_v, _, _, osem = S
    def step(s, c2):
      sl = pl.ds(s * LANES, LANES)
      dx = xi_v[sl] - xj_v[sl] - ox_v[sl]
      dy = yi_v[sl] - yj_v[sl] - oy_v[sl]
      dz = zi_v[sl] - zj_v[sl] - oz_v[sl]
      d2 = jnp.maximum(dx * dx + dy * dy + dz * dz, 0.0)
      seed = plsc.bitcast(0x5F3759DF - (plsc.bitcast(d2, jnp.int32) >> 1),
                          jnp.float32)
      h = 0.5 * d2
      r = seed * (1.5 - h * seed * seed)
      r = r * (1.5 - h * r * r)
      out_v[sl] = d2 * r
      return c2
    lax.fori_loop(0, STEPS, step, 0)
    pltpu.async_copy(out_v, out_hbm.at[pl.ds(chunk_base(g), C)], osem)

  pltpu.async_copy(out_hbm.at[pl.ds(0, C)], A[5], A[8])
  pltpu.async_copy(out_hbm.at[pl.ds(0, C)], B[5], B[8])

  lin_start(0, A)
  lin_wait(A)
  gather_fire(A)
  lin_start(1, B)

  def pair(t, carry):
    g = 2 * t
    gather_drain(A)
    lin_wait(B)
    gather_fire(B)
    out_wait(A)
    compute_store(g, A)
    lin_start(g + 2, A)
    gather_drain(B)
    @pl.when(g + 2 < n_chunks)
    def _():
      lin_wait(A)
      gather_fire(A)
    out_wait(B)
    compute_store(g + 1, B)
    lin_start(g + 3, B)
    return carry

  lax.fori_loop(0, (n_chunks - 1) // 2, pair, 0)

  gather_drain(A)
  out_wait(A)
  compute_store(n_chunks - 1, A)
  out_wait(A)
  out_wait(B)


def kernel(Ra, idx_i, idx_j, offsets):
  n = Ra.shape[0]
  e = idx_i.shape[0]
  xs = Ra[:, 0]
  ys = Ra[:, 1]
  zs = Ra[:, 2]

  ox = offsets[:, 0]
  oy = offsets[:, 1]
  oz = offsets[:, 2]

  mesh = plsc.VectorSubcoreMesh(core_axis_name="c", subcore_axis_name="s")
  buf_set = [
      pltpu.VMEM((C,), jnp.int32),
      pltpu.VMEM((C,), jnp.int32),
      pltpu.VMEM((C,), jnp.float32),
      pltpu.VMEM((C,), jnp.float32),
      pltpu.VMEM((C,), jnp.float32),
      pltpu.VMEM((C,), jnp.float32),
      pltpu.VMEM((C,), jnp.float32),
      pltpu.VMEM((C,), jnp.float32),
      pltpu.VMEM((C,), jnp.float32),
      pltpu.VMEM((C,), jnp.float32),
      pltpu.VMEM((C,), jnp.float32),
      pltpu.VMEM((C,), jnp.float32),
      pltpu.SemaphoreType.DMA,
      pltpu.SemaphoreType.DMA,
      pltpu.SemaphoreType.DMA,
  ]
  run = pl.kernel(
      _body,
      out_type=jax.ShapeDtypeStruct((e,), jnp.float32),
      mesh=mesh,
      compiler_params=pltpu.CompilerParams(needs_layout_passes=False,
                                           use_tc_tiling_on_sc=False),
      scratch_types=[
          pltpu.VMEM_SHARED((n,), jnp.float32),
          pltpu.VMEM_SHARED((n,), jnp.float32),
          pltpu.VMEM_SHARED((n,), jnp.float32),
          *buf_set,
          *buf_set,
      ],
  )
  return run(xs, ys, zs, idx_i.astype(jnp.int32), idx_j.astype(jnp.int32),
             ox, oy, oz)
